# baseline (speedup 1.0000x reference)
import jax
import jax.numpy as jnp
from jax.experimental import pallas as pl
from jax.experimental.pallas import tpu as pltpu

_N_LAT, _N_LON, _N_MESH, _N_NEI = 64, 128, 128, 4
_IN_DIM, _HID, _OUT_CH = 256, 256, 78
_BATCH = 32
_N_GRID = _N_LAT * _N_LON


def _fused_body(nf_ref, w1_ref, b1_ref, w2_ref, b2_ref, idx_ref, wts_ref,
                out_ref, mesh_s, a_s):
    s = pl.program_id(0)

    @pl.when(s == 0)
    def _prep():
        for c in range(4):
            x = nf_ref[c * 8:(c + 1) * 8].reshape(8 * _N_MESH, _IN_DIM)
            h = jnp.dot(x, w1_ref[...],
                        preferred_element_type=jnp.float32)
            h = jnp.maximum(h + b1_ref[...], 0.0)
            o = jnp.dot(h, w2_ref[...],
                        preferred_element_type=jnp.float32)
            o = o + b2_ref[...]
            mesh_s[c * 8:(c + 1) * 8] = (
                o.reshape(8, _N_MESH, _OUT_CH).astype(jnp.bfloat16))

        iota = jax.lax.broadcasted_iota(jnp.int32, (_N_GRID // 4, _N_MESH), 1)
        for c in range(4):
            rows = pl.ds(c * (_N_GRID // 4), _N_GRID // 4)
            acc = jnp.zeros((_N_GRID // 4, _N_MESH), jnp.float32)
            for k in range(_N_NEI):
                acc = acc + jnp.where(idx_ref[rows, k:k + 1] == iota,
                                      wts_ref[rows, k:k + 1], 0.0)
            a_s[rows] = acc.astype(jnp.bfloat16)

    @pl.when(s > 0)
    def _interp():
        b = s - 1
        r = jax.lax.dot_general(
            a_s[...], mesh_s[b],
            (((1,), (0,)), ((), ())),
            preferred_element_type=jnp.float32)
        out_ref[...] = r.reshape(1, _N_LAT, _N_LON, _OUT_CH)


def kernel(node_features, W1, b1, W2, b2, neighbor_indices, neighbor_weights):
    out = pl.pallas_call(
        _fused_body,
        grid=(_BATCH + 1,),
        in_specs=[
            pl.BlockSpec((_BATCH, _N_MESH, _IN_DIM), lambda s: (0, 0, 0)),
            pl.BlockSpec((_IN_DIM, _HID), lambda s: (0, 0)),
            pl.BlockSpec((1, _HID), lambda s: (0, 0)),
            pl.BlockSpec((_HID, _OUT_CH), lambda s: (0, 0)),
            pl.BlockSpec((1, _OUT_CH), lambda s: (0, 0)),
            pl.BlockSpec((_N_GRID, _N_NEI), lambda s: (0, 0)),
            pl.BlockSpec((_N_GRID, _N_NEI), lambda s: (0, 0)),
        ],
        out_specs=pl.BlockSpec((1, _N_LAT, _N_LON, _OUT_CH),
                               lambda s: (jnp.maximum(s - 1, 0), 0, 0, 0)),
        out_shape=jax.ShapeDtypeStruct((_BATCH, _N_LAT, _N_LON, _OUT_CH),
                                       jnp.float32),
        scratch_shapes=[
            pltpu.VMEM((_BATCH, _N_MESH, _OUT_CH), jnp.bfloat16),
            pltpu.VMEM((_N_GRID, _N_MESH), jnp.bfloat16),
        ],
        compiler_params=pltpu.CompilerParams(
            dimension_semantics=("arbitrary",)),
    )(node_features, W1, b1.reshape(1, _HID), W2, b2.reshape(1, _OUT_CH),
      neighbor_indices, neighbor_weights)

    return out
